# baseline (speedup 1.0000x reference)
import functools

import jax
import jax.numpy as jnp
from jax import lax
from jax.experimental import pallas as pl
from jax.experimental.pallas import tpu as pltpu
from jax.experimental.pallas import tpu_sc as plsc

NUM_FRAMES = 100000
J = 24
B = 4096
DIN = J * 3
DOUT = J * 4
FPB = 25600
NBLK = (NUM_FRAMES + FPB - 1) // FPB
NROWPAD = NBLK * FPB

NC = 2
NS = 16
NW = NC * NS
L = 16
BPW = B // NW

_MESH = plsc.VectorSubcoreMesh(core_axis_name="c", subcore_axis_name="s")


def _transpose_tc(x_ref, o_ref):
    x = x_ref[...]
    xq, yq, zq = x[0:J], x[J:2 * J], x[2 * J:3 * J]
    n2 = xq * xq + yq * yq + zq * zq
    w = jnp.sqrt(jnp.maximum(1.0 - n2, 0.0))
    out = jnp.stack([w, xq, yq, zq], axis=1).reshape(DOUT, FPB)
    o_ref[:, :, 0:DOUT] = out.T.reshape(FPB // 128, 128, DOUT)


@functools.partial(
    pl.kernel,
    mesh=_MESH,
    out_type=jax.ShapeDtypeStruct((B, 128), jnp.float32),
    scratch_types=[
        pltpu.VMEM((BPW,), jnp.int32),
        pltpu.VMEM((BPW, 128), jnp.float32),
        pltpu.SemaphoreType.DMA,
    ],
)
def _gather_sc(tab_hbm, fid_hbm, out_hbm, fid_v, rows_v, sem):
    wid = lax.axis_index("s") * NC + lax.axis_index("c")
    base = wid * BPW
    pltpu.sync_copy(fid_hbm.at[pl.ds(base, BPW)], fid_v)
    pltpu.async_copy(tab_hbm.at[fid_v], rows_v, sem).wait()
    pltpu.sync_copy(rows_v, out_hbm.at[pl.ds(base, BPW)])


def _quat_tc(g_ref, o_ref):
    lane = lax.broadcasted_iota(jnp.int32, (B, 128), 1)
    g2 = jnp.where(lane < DIN, g_ref[...], 0.0)
    r = lax.broadcasted_iota(jnp.int32, (128, DOUT), 0)
    c = lax.broadcasted_iota(jnp.int32, (128, DOUT), 1)
    cm = c % 4
    pmat = ((cm != 0) & (r == (cm - 1) * J + c // 4)).astype(jnp.float32)
    smat = ((cm == 0) & (r < DIN) & (r % J == c // 4)).astype(jnp.float32)
    dn = (((0,), (1,)), ((), ()))
    quat_rows = lax.dot_general(pmat, g2, dn, precision=lax.Precision.HIGHEST)
    n2 = lax.dot_general(smat, g2 * g2, dn, precision=lax.Precision.HIGHEST)
    ws = jnp.sqrt(jnp.maximum(1.0 - n2, 0.0))
    wmask = lax.broadcasted_iota(jnp.int32, (DOUT, B), 0) % 4 == 0
    out_t = quat_rows + jnp.where(wmask, ws, 0.0)
    o_ref[...] = out_t.reshape(J, 4, B)


def kernel(pose, frame_id):
    pose_t = pose.transpose(2, 1, 0).reshape(DIN, NUM_FRAMES)
    tab = pl.pallas_call(
        _transpose_tc,
        grid=(NBLK,),
        in_specs=[pl.BlockSpec((DIN, FPB), lambda i: (0, i))],
        out_specs=pl.BlockSpec(
            (FPB // 128, 128, 128), lambda i: (i, 0, 0)
        ),
        out_shape=jax.ShapeDtypeStruct((NROWPAD // 128, 128, 128), jnp.float32),
    )(pose_t)
    g = _gather_sc(tab.reshape(NROWPAD, 128), frame_id)
    return g[:, :DOUT].reshape(B, J, 4)


# speedup vs baseline: 1.2844x; 1.2844x over previous
import functools

import jax
import jax.numpy as jnp
from jax import lax
from jax.experimental import pallas as pl
from jax.experimental.pallas import tpu as pltpu
from jax.experimental.pallas import tpu_sc as plsc

NUM_FRAMES = 100000
J = 24
B = 4096
DIN = J * 3
DOUT = J * 4
FPB = 25600
NBLK = (NUM_FRAMES + FPB - 1) // FPB
NROWPAD = NBLK * FPB

NC = 2
NS = 16
NW = NC * NS
L = 16
BPW = B // NW

_MESH = plsc.VectorSubcoreMesh(core_axis_name="c", subcore_axis_name="s")


def _transpose_tc(x_ref, o_ref):
    x = x_ref[...]
    o_ref[:, :, 0:DIN] = x.T.reshape(FPB // 128, 128, DIN)


@functools.partial(
    pl.kernel,
    mesh=_MESH,
    out_type=jax.ShapeDtypeStruct((B, 128), jnp.float32),
    scratch_types=[
        pltpu.VMEM((BPW,), jnp.int32),
        pltpu.VMEM((BPW, 128), jnp.float32),
        pltpu.SemaphoreType.DMA,
    ],
)
def _gather_sc(tab_hbm, fid_hbm, out_hbm, fid_v, rows_v, sem):
    wid = lax.axis_index("s") * NC + lax.axis_index("c")
    base = wid * BPW
    pltpu.sync_copy(fid_hbm.at[pl.ds(base, BPW)], fid_v)
    pltpu.async_copy(tab_hbm.at[fid_v], rows_v, sem).wait()
    pltpu.sync_copy(rows_v, out_hbm.at[pl.ds(base, BPW)])


def _quat_tc(g_ref, o_ref):
    lane = lax.broadcasted_iota(jnp.int32, (B, 128), 1)
    g2 = jnp.where(lane < DIN, g_ref[...], 0.0)
    r = lax.broadcasted_iota(jnp.int32, (128, DOUT), 0)
    c = lax.broadcasted_iota(jnp.int32, (128, DOUT), 1)
    cm = c % 4
    pmat = ((cm != 0) & (r == (cm - 1) * J + c // 4)).astype(jnp.float32)
    smat = ((cm == 0) & (r < DIN) & (r % J == c // 4)).astype(jnp.float32)
    dn = (((0,), (1,)), ((), ()))
    quat_rows = lax.dot_general(pmat, g2, dn, precision=lax.Precision.HIGHEST)
    n2 = lax.dot_general(smat, g2 * g2, dn, precision=lax.Precision.HIGHEST)
    ws = jnp.sqrt(jnp.maximum(1.0 - n2, 0.0))
    wmask = lax.broadcasted_iota(jnp.int32, (DOUT, B), 0) % 4 == 0
    out_t = quat_rows + jnp.where(wmask, ws, 0.0)
    o_ref[...] = out_t.reshape(J, 4, B)


def kernel(pose, frame_id):
    pose_t = pose.transpose(2, 1, 0).reshape(DIN, NUM_FRAMES)
    tab = pl.pallas_call(
        _transpose_tc,
        grid=(NBLK,),
        in_specs=[pl.BlockSpec((DIN, FPB), lambda i: (0, i))],
        out_specs=pl.BlockSpec(
            (FPB // 128, 128, 128), lambda i: (i, 0, 0)
        ),
        out_shape=jax.ShapeDtypeStruct((NROWPAD // 128, 128, 128), jnp.float32),
    )(pose_t)
    g = _gather_sc(tab.reshape(NROWPAD, 128), frame_id)
    q_t = pl.pallas_call(
        _quat_tc,
        out_shape=jax.ShapeDtypeStruct((J, 4, B), jnp.float32),
    )(g)
    return q_t.transpose(2, 0, 1)
